# baseline (speedup 1.0000x reference)
import jax
import jax.numpy as jnp
from jax import lax
from jax.experimental import pallas as pl
from jax.experimental.pallas import tpu as pltpu

SEQ = 8192
EMB = 10
HID = 26
VOCAB = 26
NCLS = 26


def _fused_kernel(x_ref, emb_ref, wih_t_ref, whh_ref, wfc_t_ref,
                  bih_ref, bhh_ref, bfc_ref, out_ref, t2s_ref, h_ref):
    t2s_ref[...] = jnp.dot(emb_ref[...], wih_t_ref[...],
                           preferred_element_type=jnp.float32) + bih_ref[...]

    whh = whh_ref[...]
    bhh = bhh_ref[...]

    def body(t, h):
        v = x_ref[t]
        a = t2s_ref[pl.ds(v, 1), :]
        hw = lax.dot_general(h, whh, (((1,), (1,)), ((), ())),
                             preferred_element_type=jnp.float32)
        hn = jnp.tanh((a + hw) + bhh)
        h_ref[pl.ds(t, 1), :] = hn
        return hn

    lax.fori_loop(0, SEQ, body, jnp.zeros((1, HID), jnp.float32), unroll=8)

    out_ref[...] = jnp.dot(h_ref[...], wfc_t_ref[...],
                           preferred_element_type=jnp.float32) + bfc_ref[...]


def kernel(x, emb_table, W_ih, W_hh, b_ih, b_hh, W_fc, b_fc):
    xr = x.reshape(SEQ).astype(jnp.int32)
    out = pl.pallas_call(
        _fused_kernel,
        out_shape=jax.ShapeDtypeStruct((SEQ, NCLS), jnp.float32),
        in_specs=[pl.BlockSpec(memory_space=pltpu.SMEM),
                  pl.BlockSpec(), pl.BlockSpec(), pl.BlockSpec(),
                  pl.BlockSpec(), pl.BlockSpec(), pl.BlockSpec(),
                  pl.BlockSpec()],
        scratch_shapes=[pltpu.VMEM((VOCAB, HID), jnp.float32),
                        pltpu.VMEM((SEQ, HID), jnp.float32)],
    )(xr, emb_table, W_ih.T, W_hh, W_fc.T, b_ih.reshape(1, HID),
      b_hh.reshape(1, HID), b_fc.reshape(1, NCLS))
    return out.reshape(1, SEQ, NCLS)


# speedup vs baseline: 1.0051x; 1.0051x over previous
import jax
import jax.numpy as jnp
from jax import lax
from jax.experimental import pallas as pl
from jax.experimental.pallas import tpu as pltpu

SEQ = 8192
EMB = 10
HID = 26
VOCAB = 26
NCLS = 26


def _fused_kernel(x_ref, emb_ref, wih_t_ref, whh_ref, wfc_t_ref,
                  bih_ref, bhh_ref, bfc_ref, out_ref, t2s_ref, h_ref):
    t2s_ref[...] = jnp.dot(emb_ref[...], wih_t_ref[...],
                           preferred_element_type=jnp.float32) + bih_ref[...]

    whh = whh_ref[...]
    bhh = bhh_ref[...]

    def body(t, h):
        v = x_ref[t]
        a = t2s_ref[pl.ds(v, 1), :]
        hw = lax.dot_general(h, whh, (((1,), (1,)), ((), ())),
                             preferred_element_type=jnp.float32)
        hn = jnp.tanh((a + hw) + bhh)
        h_ref[pl.ds(t, 1), :] = hn
        return hn

    lax.fori_loop(0, SEQ, body, jnp.zeros((1, HID), jnp.float32), unroll=16)

    out_ref[...] = jnp.dot(h_ref[...], wfc_t_ref[...],
                           preferred_element_type=jnp.float32) + bfc_ref[...]


def kernel(x, emb_table, W_ih, W_hh, b_ih, b_hh, W_fc, b_fc):
    xr = x.reshape(SEQ).astype(jnp.int32)
    out = pl.pallas_call(
        _fused_kernel,
        out_shape=jax.ShapeDtypeStruct((SEQ, NCLS), jnp.float32),
        in_specs=[pl.BlockSpec(memory_space=pltpu.SMEM),
                  pl.BlockSpec(), pl.BlockSpec(), pl.BlockSpec(),
                  pl.BlockSpec(), pl.BlockSpec(), pl.BlockSpec(),
                  pl.BlockSpec()],
        scratch_shapes=[pltpu.VMEM((VOCAB, HID), jnp.float32),
                        pltpu.VMEM((SEQ, HID), jnp.float32)],
    )(xr, emb_table, W_ih.T, W_hh, W_fc.T, b_ih.reshape(1, HID),
      b_hh.reshape(1, HID), b_fc.reshape(1, NCLS))
    return out.reshape(1, SEQ, NCLS)
